# baseline (speedup 1.0000x reference)
import functools

import jax
import jax.numpy as jnp
from jax import lax
from jax.experimental import pallas as pl
from jax.experimental.pallas import tpu as pltpu
from jax.experimental.pallas import tpu_sc as plsc

_NP = 10000
_NC = 40000
_NT = _NP + _NC
_E = 800000
_H = 64
_HH = 32
_CW = 16

_R = 50016
_TG = 396
_BLK = 3
_NBLK = _TG // _BLK
_EP = 16 * _TG * 128
_RPT = _R // 16


def _sc_mesh():
    return plsc.VectorSubcoreMesh(
        core_axis_name="c", subcore_axis_name="s", num_cores=2, num_subcores=16
    )


def _make_sc_segsum():
    blk_e = _BLK * 128
    out_type = jax.ShapeDtypeStruct((2, _R, _HH), jnp.float32)
    scratch = [
        pltpu.VMEM((blk_e,), jnp.int32),
        pltpu.VMEM((blk_e,), jnp.int32),
        pltpu.VMEM((blk_e,), jnp.int32),
        pltpu.VMEM((blk_e,), jnp.int32),
        pltpu.VMEM((blk_e, _HH), jnp.float32),
        pltpu.VMEM((blk_e, _HH), jnp.float32),
        pltpu.VMEM_SHARED((_R, _HH), jnp.float32),
        pltpu.SemaphoreType.DMA,
        pltpu.SemaphoreType.DMA,
        pltpu.SemaphoreType.DMA,
        pltpu.SemaphoreType.DMA,
    ]

    def body(h_ref, srcb_ref, dstb_ref, z32_ref, acc_out,
             src0, dst0, src1, dst1, rows0, rows1, acc_sp,
             semg0, semg1, sems0, sems1):
        c = lax.axis_index("c")
        s = lax.axis_index("s")
        rb = s * _RPT

        pltpu.sync_copy(z32_ref, acc_sp.at[pl.ds(rb, _RPT)])
        plsc.subcore_barrier()

        bbase = s * _NBLK

        def load(src_b, dst_b, blk):
            pltpu.sync_copy(srcb_ref.at[blk], src_b)
            pltpu.sync_copy(dstb_ref.at[blk], dst_b)
            for j in range(blk_e // 16):
                v = src_b[pl.ds(j * 16, 16)]
                src_b[pl.ds(j * 16, 16)] = v * 2 + c

        def blk_body(i, carry):
            load(src0, dst0, bbase + 2 * i)
            g0 = pltpu.async_copy(h_ref.at[src0], rows0, semg0)
            load(src1, dst1, bbase + 2 * i + 1)
            g1 = pltpu.async_copy(h_ref.at[src1], rows1, semg1)
            g0.wait()
            s0 = pltpu.async_copy(rows0, acc_sp.at[dst0], sems0, add=True)
            g1.wait()
            s1 = pltpu.async_copy(rows1, acc_sp.at[dst1], sems1, add=True)
            s0.wait()
            s1.wait()
            return carry

        lax.fori_loop(0, _NBLK // 2, blk_body, 0)
        plsc.subcore_barrier()

        pltpu.sync_copy(acc_sp.at[pl.ds(rb, _RPT)], acc_out.at[c, pl.ds(rb, _RPT)])

    return pl.kernel(
        body, out_type=out_type, mesh=_sc_mesh(), scratch_types=scratch,
        compiler_params=pltpu.CompilerParams(use_tc_tiling_on_sc=False),
    )


def _make_sc_counts():
    blk_e = _BLK * 128
    out_type = jax.ShapeDtypeStruct((2, _R, _CW), jnp.float32)
    scratch = [
        pltpu.VMEM((blk_e,), jnp.int32),
        pltpu.VMEM((blk_e, _CW), jnp.float32),
        pltpu.VMEM_SHARED((_R, _CW), jnp.float32),
    ]
    half_b = _NBLK // 2

    def body(dstb_ref, zc_ref, ones_hbm, cnt_out, dst_b, ones_b, cnt_sp):
        c = lax.axis_index("c")
        s = lax.axis_index("s")
        rb = s * _RPT

        pltpu.sync_copy(zc_ref, cnt_sp.at[pl.ds(rb, _RPT)])
        pltpu.sync_copy(ones_hbm, ones_b)
        plsc.subcore_barrier()

        bbase = s * _NBLK + c * half_b

        def blk_body(b, carry):
            pltpu.sync_copy(dstb_ref.at[bbase + b], dst_b)
            pltpu.sync_copy(ones_b, cnt_sp.at[dst_b], add=True)
            return carry

        lax.fori_loop(0, half_b, blk_body, 0)
        plsc.subcore_barrier()

        pltpu.sync_copy(cnt_sp.at[pl.ds(rb, _RPT)], cnt_out.at[c, pl.ds(rb, _RPT)])

    return pl.kernel(
        body, out_type=out_type, mesh=_sc_mesh(), scratch_types=scratch,
        compiler_params=pltpu.CompilerParams(use_tc_tiling_on_sc=False),
    )


_sc_segsum_plain = _make_sc_segsum()
_sc_counts = _make_sc_counts()


_BP = 1000


_NPB = _NP // _BP


def _proj_body(xp_ref, pidx_ref, xc_ref, sidx_ref, iidx_ref,
               ptab_ref, stab_ref, itab_ref,
               wp_ref, bp_ref, wc_ref, bc_ref, o_ref):
    pid = pl.program_id(0)

    @pl.when(pid < _NPB)
    def _():
        idx = pidx_ref[0, 0, :]
        oh = (idx[:, None] == lax.broadcasted_iota(jnp.int32, (_BP, 50), 1)
              ).astype(jnp.float32)
        emb = jnp.dot(oh, ptab_ref[...], preferred_element_type=jnp.float32)
        feat = jnp.concatenate([xp_ref[...], emb], axis=1)
        h = (jnp.dot(feat, wp_ref[...], preferred_element_type=jnp.float32)
             + bp_ref[...])
        o_ref[...] = jnp.maximum(h, 0.0)

    @pl.when(pid >= _NPB)
    def _():
        sidx = sidx_ref[0, 0, :]
        iidx = iidx_ref[0, 0, :]
        soh = (sidx[:, None] == lax.broadcasted_iota(jnp.int32, (_BP, 12), 1)
               ).astype(jnp.float32)
        ioh = (iidx[:, None] == lax.broadcasted_iota(jnp.int32, (_BP, 150), 1)
               ).astype(jnp.float32)
        semb = jnp.dot(soh, stab_ref[...], preferred_element_type=jnp.float32)
        iemb = jnp.dot(ioh, itab_ref[...], preferred_element_type=jnp.float32)
        feat = jnp.concatenate([xc_ref[...], semb, iemb], axis=1)
        h = (jnp.dot(feat, wc_ref[...], preferred_element_type=jnp.float32)
             + bc_ref[...])
        o_ref[...] = jnp.maximum(h, 0.0)


def _proj(xp, pidx3, xc, sidx3, iidx3, ptab, stab, itab, wp_t, bp2, wc_t, bc2):
    pmap = lambda i: (jnp.minimum(i, _NPB - 1), 0)
    pmap3 = lambda i: (jnp.minimum(i, _NPB - 1), 0, 0)
    cmap = lambda i: (jnp.maximum(i - _NPB, 0), 0)
    cmap3 = lambda i: (jnp.maximum(i - _NPB, 0), 0, 0)
    full = lambda i: (0, 0)
    return pl.pallas_call(
        _proj_body,
        grid=(_NT // _BP,),
        in_specs=[
            pl.BlockSpec((_BP, 56), pmap),
            pl.BlockSpec((1, 1, _BP), pmap3),
            pl.BlockSpec((_BP, 48), cmap),
            pl.BlockSpec((1, 1, _BP), cmap3),
            pl.BlockSpec((1, 1, _BP), cmap3),
            pl.BlockSpec((50, 8), full),
            pl.BlockSpec((12, 8), full),
            pl.BlockSpec((150, 8), full),
            pl.BlockSpec((_H, _H), full),
            pl.BlockSpec((1, _H), full),
            pl.BlockSpec((_H, _H), full),
            pl.BlockSpec((1, _H), full),
        ],
        out_specs=pl.BlockSpec((_BP, _H), lambda i: (i, 0)),
        out_shape=jax.ShapeDtypeStruct((_NT, _H), jnp.float32),
    )(xp, pidx3, xc, sidx3, iidx3, ptab, stab, itab, wp_t, bp2, wc_t, bc2)


def _layer_body(acc_ref, cnt_ref, h_ref, wl_ref, b_ref, wr_ref, o_ref, *,
                relu):
    cnt = cnt_ref[0] + cnt_ref[1]
    inv = 1.0 / jnp.maximum(cnt[:, :1], 1.0)
    mean = jnp.concatenate([acc_ref[0], acc_ref[1]], axis=1) * inv
    y = (jnp.dot(mean, wl_ref[...], preferred_element_type=jnp.float32)
         + b_ref[...]
         + jnp.dot(h_ref[...], wr_ref[...], preferred_element_type=jnp.float32))
    if relu:
        y = jnp.maximum(y, 0.0)
    o_ref[...] = y


def _layer(acc, cnt, h, wl_t, b2, wr_t, relu):
    grid = _NT // _BP
    return pl.pallas_call(
        functools.partial(_layer_body, relu=relu),
        grid=(grid,),
        in_specs=[
            pl.BlockSpec((2, _BP, _HH), lambda i: (0, i, 0)),
            pl.BlockSpec((2, _BP, _CW), lambda i: (0, i, 0)),
            pl.BlockSpec((_BP, _H), lambda i: (i, 0)),
            pl.BlockSpec((_H, _H), lambda i: (0, 0)),
            pl.BlockSpec((1, _H), lambda i: (0, 0)),
            pl.BlockSpec((_H, _H), lambda i: (0, 0)),
        ],
        out_specs=pl.BlockSpec((_BP, _H), lambda i: (i, 0)),
        out_shape=jax.ShapeDtypeStruct((_NT, _H), jnp.float32),
    )(acc, cnt, h, wl_t, b2, wr_t)


def kernel(x_pol_dyn, pol_state_idx, x_comp_dyn, comp_sector_idx, comp_ind_idx,
           edge_index, state_emb_table, sector_emb_table, ind_emb_table,
           W_pol, b_pol, W_comp, b_comp, W1_l, b1, W1_r, W2_l, b2, W2_r):
    f32 = jnp.float32
    i32 = jnp.int32

    pol_idx3 = pol_state_idx.astype(i32).reshape(_NP // _BP, 1, _BP)
    sec_idx3 = comp_sector_idx.astype(i32).reshape(_NC // _BP, 1, _BP)
    ind_idx3 = comp_ind_idx.astype(i32).reshape(_NC // _BP, 1, _BP)

    h = _proj(x_pol_dyn.astype(f32), pol_idx3, x_comp_dyn.astype(f32),
              sec_idx3, ind_idx3, state_emb_table, sector_emb_table,
              ind_emb_table, W_pol.T.astype(f32), b_pol.reshape(1, _H),
              W_comp.T.astype(f32), b_comp.reshape(1, _H))

    src = edge_index[0].astype(i32)
    dst = edge_index[1].astype(i32)
    pad = _EP - _E
    srcb = jnp.concatenate([src, jnp.zeros((pad,), i32)]).reshape(-1, _BLK * 128)
    dstb = jnp.concatenate([dst, jnp.full((pad,), _NT, i32)]).reshape(-1, _BLK * 128)

    z32 = jnp.zeros((_RPT, _HH), f32)
    zc = jnp.zeros((_RPT, _CW), f32)
    ones = jnp.ones((_BLK * 128, _CW), f32)

    cnt = _sc_counts(dstb, zc, ones)
    acc1 = _sc_segsum_plain(h.reshape(-1, _HH), srcb, dstb, z32)
    h1 = _layer(acc1, cnt, h, W1_l.T.astype(f32), b1.reshape(1, _H),
                W1_r.T.astype(f32), relu=True)

    acc2 = _sc_segsum_plain(h1.reshape(-1, _HH), srcb, dstb, z32)
    h2 = _layer(acc2, cnt, h1, W2_l.T.astype(f32), b2.reshape(1, _H),
                W2_r.T.astype(f32), relu=False)

    return (h2[:_NP], h2[_NP:])


# speedup vs baseline: 1.1637x; 1.1637x over previous
import functools

import jax
import jax.numpy as jnp
from jax import lax
from jax.experimental import pallas as pl
from jax.experimental.pallas import tpu as pltpu
from jax.experimental.pallas import tpu_sc as plsc

_NP = 10000
_NC = 40000
_NT = _NP + _NC
_E = 800000
_H = 64
_HH = 32
_CW = 16

_R = 50016
_TG = 392
_BLK = 4
_NBLK = _TG // _BLK
_EP = 16 * _TG * 128
_RPT = _R // 16


def _sc_mesh():
    return plsc.VectorSubcoreMesh(
        core_axis_name="c", subcore_axis_name="s", num_cores=2, num_subcores=16
    )


def _make_sc_segsum():
    out_type = jax.ShapeDtypeStruct((2, _R, _HH), jnp.float32)
    scratch = [
        pltpu.VMEM((_BLK, 128), jnp.int32),
        pltpu.VMEM((_BLK, 128), jnp.int32),
        pltpu.VMEM((_BLK, 128, _HH), jnp.float32),
        pltpu.VMEM_SHARED((_R, _HH), jnp.float32),
        pltpu.SemaphoreType.DMA,
        pltpu.SemaphoreType.DMA,
    ]

    def body(h_ref, srcb_ref, dstb_ref, z32_ref, acc_out,
             src_b, dst_b, rows_b, acc_sp, semg, sems):
        c = lax.axis_index("c")
        s = lax.axis_index("s")
        rb = s * _RPT

        pltpu.sync_copy(z32_ref, acc_sp.at[pl.ds(rb, _RPT)])
        plsc.subcore_barrier()

        gbase = s * _TG

        def blk_body(b, carry):
            g0 = gbase + b * _BLK
            pltpu.sync_copy(srcb_ref.at[pl.ds(g0, _BLK)], src_b)
            pltpu.sync_copy(dstb_ref.at[pl.ds(g0, _BLK)], dst_b)
            for gi in range(_BLK):
                for j in range(8):
                    v = src_b[gi, pl.ds(j * 16, 16)]
                    src_b[gi, pl.ds(j * 16, 16)] = v * 2 + c
            gs = [
                pltpu.async_copy(h_ref.at[src_b.at[gi]], rows_b.at[gi], semg)
                for gi in range(_BLK)
            ]
            for cp in gs:
                cp.wait()
            ss = [
                pltpu.async_copy(
                    rows_b.at[gi], acc_sp.at[dst_b.at[gi]], sems, add=True)
                for gi in range(_BLK)
            ]
            for cp in ss:
                cp.wait()
            return carry

        lax.fori_loop(0, _NBLK, blk_body, 0)
        plsc.subcore_barrier()

        pltpu.sync_copy(acc_sp.at[pl.ds(rb, _RPT)], acc_out.at[c, pl.ds(rb, _RPT)])

    return pl.kernel(
        body, out_type=out_type, mesh=_sc_mesh(), scratch_types=scratch,
        compiler_params=pltpu.CompilerParams(use_tc_tiling_on_sc=False),
    )


def _make_sc_counts():
    blk_e = _BLK * 128
    out_type = jax.ShapeDtypeStruct((2, _R, _CW), jnp.float32)
    scratch = [
        pltpu.VMEM((blk_e,), jnp.int32),
        pltpu.VMEM((blk_e, _CW), jnp.float32),
        pltpu.VMEM_SHARED((_R, _CW), jnp.float32),
    ]
    half_b = _NBLK // 2

    def body(dstb_ref, zc_ref, ones_hbm, cnt_out, dst_b, ones_b, cnt_sp):
        c = lax.axis_index("c")
        s = lax.axis_index("s")
        rb = s * _RPT

        pltpu.sync_copy(zc_ref, cnt_sp.at[pl.ds(rb, _RPT)])
        pltpu.sync_copy(ones_hbm, ones_b)
        plsc.subcore_barrier()

        bbase = s * _NBLK + c * half_b

        def blk_body(b, carry):
            pltpu.sync_copy(dstb_ref.at[bbase + b], dst_b)
            pltpu.sync_copy(ones_b, cnt_sp.at[dst_b], add=True)
            return carry

        lax.fori_loop(0, half_b, blk_body, 0)
        plsc.subcore_barrier()

        pltpu.sync_copy(cnt_sp.at[pl.ds(rb, _RPT)], cnt_out.at[c, pl.ds(rb, _RPT)])

    return pl.kernel(
        body, out_type=out_type, mesh=_sc_mesh(), scratch_types=scratch,
        compiler_params=pltpu.CompilerParams(use_tc_tiling_on_sc=False),
    )


_sc_segsum_plain = _make_sc_segsum()
_sc_counts = _make_sc_counts()


_BP = 1000


_NPB = _NP // _BP


def _proj_body(xp_ref, pidx_ref, xc_ref, sidx_ref, iidx_ref,
               ptab_ref, stab_ref, itab_ref,
               wp_ref, bp_ref, wc_ref, bc_ref, o_ref):
    pid = pl.program_id(0)

    @pl.when(pid < _NPB)
    def _():
        idx = pidx_ref[0, 0, :]
        oh = (idx[:, None] == lax.broadcasted_iota(jnp.int32, (_BP, 50), 1)
              ).astype(jnp.float32)
        emb = jnp.dot(oh, ptab_ref[...], preferred_element_type=jnp.float32)
        feat = jnp.concatenate([xp_ref[...], emb], axis=1)
        h = (jnp.dot(feat, wp_ref[...], preferred_element_type=jnp.float32)
             + bp_ref[...])
        o_ref[...] = jnp.maximum(h, 0.0)

    @pl.when(pid >= _NPB)
    def _():
        sidx = sidx_ref[0, 0, :]
        iidx = iidx_ref[0, 0, :]
        soh = (sidx[:, None] == lax.broadcasted_iota(jnp.int32, (_BP, 12), 1)
               ).astype(jnp.float32)
        ioh = (iidx[:, None] == lax.broadcasted_iota(jnp.int32, (_BP, 150), 1)
               ).astype(jnp.float32)
        semb = jnp.dot(soh, stab_ref[...], preferred_element_type=jnp.float32)
        iemb = jnp.dot(ioh, itab_ref[...], preferred_element_type=jnp.float32)
        feat = jnp.concatenate([xc_ref[...], semb, iemb], axis=1)
        h = (jnp.dot(feat, wc_ref[...], preferred_element_type=jnp.float32)
             + bc_ref[...])
        o_ref[...] = jnp.maximum(h, 0.0)


def _proj(xp, pidx3, xc, sidx3, iidx3, ptab, stab, itab, wp_t, bp2, wc_t, bc2):
    pmap = lambda i: (jnp.minimum(i, _NPB - 1), 0)
    pmap3 = lambda i: (jnp.minimum(i, _NPB - 1), 0, 0)
    cmap = lambda i: (jnp.maximum(i - _NPB, 0), 0)
    cmap3 = lambda i: (jnp.maximum(i - _NPB, 0), 0, 0)
    full = lambda i: (0, 0)
    return pl.pallas_call(
        _proj_body,
        grid=(_NT // _BP,),
        in_specs=[
            pl.BlockSpec((_BP, 56), pmap),
            pl.BlockSpec((1, 1, _BP), pmap3),
            pl.BlockSpec((_BP, 48), cmap),
            pl.BlockSpec((1, 1, _BP), cmap3),
            pl.BlockSpec((1, 1, _BP), cmap3),
            pl.BlockSpec((50, 8), full),
            pl.BlockSpec((12, 8), full),
            pl.BlockSpec((150, 8), full),
            pl.BlockSpec((_H, _H), full),
            pl.BlockSpec((1, _H), full),
            pl.BlockSpec((_H, _H), full),
            pl.BlockSpec((1, _H), full),
        ],
        out_specs=pl.BlockSpec((_BP, _H), lambda i: (i, 0)),
        out_shape=jax.ShapeDtypeStruct((_NT, _H), jnp.float32),
    )(xp, pidx3, xc, sidx3, iidx3, ptab, stab, itab, wp_t, bp2, wc_t, bc2)


def _layer_body(acc_ref, cnt_ref, h_ref, wl_ref, b_ref, wr_ref, o_ref, *,
                relu):
    cnt = cnt_ref[0] + cnt_ref[1]
    inv = 1.0 / jnp.maximum(cnt[:, :1], 1.0)
    mean = jnp.concatenate([acc_ref[0], acc_ref[1]], axis=1) * inv
    y = (jnp.dot(mean, wl_ref[...], preferred_element_type=jnp.float32)
         + b_ref[...]
         + jnp.dot(h_ref[...], wr_ref[...], preferred_element_type=jnp.float32))
    if relu:
        y = jnp.maximum(y, 0.0)
    o_ref[...] = y


def _layer(acc, cnt, h, wl_t, b2, wr_t, relu):
    grid = _NT // _BP
    return pl.pallas_call(
        functools.partial(_layer_body, relu=relu),
        grid=(grid,),
        in_specs=[
            pl.BlockSpec((2, _BP, _HH), lambda i: (0, i, 0)),
            pl.BlockSpec((2, _BP, _CW), lambda i: (0, i, 0)),
            pl.BlockSpec((_BP, _H), lambda i: (i, 0)),
            pl.BlockSpec((_H, _H), lambda i: (0, 0)),
            pl.BlockSpec((1, _H), lambda i: (0, 0)),
            pl.BlockSpec((_H, _H), lambda i: (0, 0)),
        ],
        out_specs=pl.BlockSpec((_BP, _H), lambda i: (i, 0)),
        out_shape=jax.ShapeDtypeStruct((_NT, _H), jnp.float32),
    )(acc, cnt, h, wl_t, b2, wr_t)


def kernel(x_pol_dyn, pol_state_idx, x_comp_dyn, comp_sector_idx, comp_ind_idx,
           edge_index, state_emb_table, sector_emb_table, ind_emb_table,
           W_pol, b_pol, W_comp, b_comp, W1_l, b1, W1_r, W2_l, b2, W2_r):
    f32 = jnp.float32
    i32 = jnp.int32

    pol_idx3 = pol_state_idx.astype(i32).reshape(_NP // _BP, 1, _BP)
    sec_idx3 = comp_sector_idx.astype(i32).reshape(_NC // _BP, 1, _BP)
    ind_idx3 = comp_ind_idx.astype(i32).reshape(_NC // _BP, 1, _BP)

    h = _proj(x_pol_dyn.astype(f32), pol_idx3, x_comp_dyn.astype(f32),
              sec_idx3, ind_idx3, state_emb_table, sector_emb_table,
              ind_emb_table, W_pol.T.astype(f32), b_pol.reshape(1, _H),
              W_comp.T.astype(f32), b_comp.reshape(1, _H))

    src = edge_index[0].astype(i32)
    dst = edge_index[1].astype(i32)
    pad = _EP - _E
    srcp = jnp.concatenate([src, jnp.zeros((pad,), i32)])
    dstp = jnp.concatenate([dst, jnp.full((pad,), _NT, i32)])
    src2 = srcp.reshape(-1, 128)
    dst2 = dstp.reshape(-1, 128)
    dstc = dstp.reshape(-1, _BLK * 128)

    z32 = jnp.zeros((_RPT, _HH), f32)
    zc = jnp.zeros((_RPT, _CW), f32)
    ones = jnp.ones((_BLK * 128, _CW), f32)

    cnt = _sc_counts(dstc, zc, ones)
    acc1 = _sc_segsum_plain(h.reshape(-1, _HH), src2, dst2, z32)
    h1 = _layer(acc1, cnt, h, W1_l.T.astype(f32), b1.reshape(1, _H),
                W1_r.T.astype(f32), relu=True)

    acc2 = _sc_segsum_plain(h1.reshape(-1, _HH), src2, dst2, z32)
    h2 = _layer(acc2, cnt, h1, W2_l.T.astype(f32), b2.reshape(1, _H),
                W2_r.T.astype(f32), relu=False)

    return (h2[:_NP], h2[_NP:])


# speedup vs baseline: 1.3058x; 1.1221x over previous
import functools

import jax
import jax.numpy as jnp
from jax import lax
from jax.experimental import pallas as pl
from jax.experimental.pallas import tpu as pltpu
from jax.experimental.pallas import tpu_sc as plsc

_NP = 10000
_NC = 40000
_NT = _NP + _NC
_E = 800000
_H = 64
_HH = 32
_CW = 16

_R = 50016
_TG = 392
_BLK = 7
_NBLK = _TG // _BLK
_EP = 16 * _TG * 128
_RPT = _R // 16


def _sc_mesh():
    return plsc.VectorSubcoreMesh(
        core_axis_name="c", subcore_axis_name="s", num_cores=2, num_subcores=16
    )


def _make_sc_segsum():
    out_type = jax.ShapeDtypeStruct((2, _R, _HH), jnp.float32)
    scratch = [
        pltpu.VMEM((_BLK, 128), jnp.int32),
        pltpu.VMEM((_BLK, 128), jnp.int32),
        pltpu.VMEM((_BLK, 128, _HH), jnp.float32),
        pltpu.VMEM_SHARED((_R, _HH), jnp.float32),
        pltpu.SemaphoreType.DMA,
        pltpu.SemaphoreType.DMA,
    ]

    def body(h_ref, srcb_ref, dstb_ref, z32_ref, acc_out,
             src_b, dst_b, rows_b, acc_sp, semg, sems):
        c = lax.axis_index("c")
        s = lax.axis_index("s")
        rb = s * _RPT

        pltpu.sync_copy(z32_ref, acc_sp.at[pl.ds(rb, _RPT)])
        plsc.subcore_barrier()

        gbase = s * _TG

        def blk_body(b, carry):
            g0 = gbase + b * _BLK
            pltpu.sync_copy(srcb_ref.at[pl.ds(g0, _BLK)], src_b)
            pltpu.sync_copy(dstb_ref.at[pl.ds(g0, _BLK)], dst_b)
            for gi in range(_BLK):
                for j in range(8):
                    v = src_b[gi, pl.ds(j * 16, 16)]
                    src_b[gi, pl.ds(j * 16, 16)] = v * 2 + c
            gs = [
                pltpu.async_copy(h_ref.at[src_b.at[gi]], rows_b.at[gi], semg)
                for gi in range(_BLK)
            ]
            for cp in gs:
                cp.wait()
            ss = [
                pltpu.async_copy(
                    rows_b.at[gi], acc_sp.at[dst_b.at[gi]], sems, add=True)
                for gi in range(_BLK)
            ]
            for cp in ss:
                cp.wait()
            return carry

        lax.fori_loop(0, _NBLK, blk_body, 0)
        plsc.subcore_barrier()

        pltpu.sync_copy(acc_sp.at[pl.ds(rb, _RPT)], acc_out.at[c, pl.ds(rb, _RPT)])

    return pl.kernel(
        body, out_type=out_type, mesh=_sc_mesh(), scratch_types=scratch,
        compiler_params=pltpu.CompilerParams(use_tc_tiling_on_sc=False),
    )


def _make_sc_counts():
    blk_e = _BLK * 128
    out_type = jax.ShapeDtypeStruct((2, _R, _CW), jnp.float32)
    scratch = [
        pltpu.VMEM((blk_e,), jnp.int32),
        pltpu.VMEM((blk_e, _CW), jnp.float32),
        pltpu.VMEM_SHARED((_R, _CW), jnp.float32),
    ]
    half_b = _NBLK // 2

    def body(dstb_ref, zc_ref, ones_hbm, cnt_out, dst_b, ones_b, cnt_sp):
        c = lax.axis_index("c")
        s = lax.axis_index("s")
        rb = s * _RPT

        pltpu.sync_copy(zc_ref, cnt_sp.at[pl.ds(rb, _RPT)])
        pltpu.sync_copy(ones_hbm, ones_b)
        plsc.subcore_barrier()

        bbase = s * _NBLK + c * half_b

        def blk_body(b, carry):
            pltpu.sync_copy(dstb_ref.at[bbase + b], dst_b)
            pltpu.sync_copy(ones_b, cnt_sp.at[dst_b], add=True)
            return carry

        lax.fori_loop(0, half_b, blk_body, 0)
        plsc.subcore_barrier()

        pltpu.sync_copy(cnt_sp.at[pl.ds(rb, _RPT)], cnt_out.at[c, pl.ds(rb, _RPT)])

    return pl.kernel(
        body, out_type=out_type, mesh=_sc_mesh(), scratch_types=scratch,
        compiler_params=pltpu.CompilerParams(use_tc_tiling_on_sc=False),
    )


_sc_segsum_plain = _make_sc_segsum()
_sc_counts = _make_sc_counts()


_BP = 1000


_NPB = _NP // _BP


def _proj_body(xp_ref, pidx_ref, xc_ref, sidx_ref, iidx_ref,
               ptab_ref, stab_ref, itab_ref,
               wp_ref, bp_ref, wc_ref, bc_ref, o_ref):
    pid = pl.program_id(0)

    @pl.when(pid < _NPB)
    def _():
        idx = pidx_ref[0, 0, :]
        oh = (idx[:, None] == lax.broadcasted_iota(jnp.int32, (_BP, 50), 1)
              ).astype(jnp.float32)
        emb = jnp.dot(oh, ptab_ref[...], preferred_element_type=jnp.float32)
        feat = jnp.concatenate([xp_ref[...], emb], axis=1)
        h = (jnp.dot(feat, wp_ref[...], preferred_element_type=jnp.float32)
             + bp_ref[...])
        o_ref[...] = jnp.maximum(h, 0.0)

    @pl.when(pid >= _NPB)
    def _():
        sidx = sidx_ref[0, 0, :]
        iidx = iidx_ref[0, 0, :]
        soh = (sidx[:, None] == lax.broadcasted_iota(jnp.int32, (_BP, 12), 1)
               ).astype(jnp.float32)
        ioh = (iidx[:, None] == lax.broadcasted_iota(jnp.int32, (_BP, 150), 1)
               ).astype(jnp.float32)
        semb = jnp.dot(soh, stab_ref[...], preferred_element_type=jnp.float32)
        iemb = jnp.dot(ioh, itab_ref[...], preferred_element_type=jnp.float32)
        feat = jnp.concatenate([xc_ref[...], semb, iemb], axis=1)
        h = (jnp.dot(feat, wc_ref[...], preferred_element_type=jnp.float32)
             + bc_ref[...])
        o_ref[...] = jnp.maximum(h, 0.0)


def _proj(xp, pidx3, xc, sidx3, iidx3, ptab, stab, itab, wp_t, bp2, wc_t, bc2):
    pmap = lambda i: (jnp.minimum(i, _NPB - 1), 0)
    pmap3 = lambda i: (jnp.minimum(i, _NPB - 1), 0, 0)
    cmap = lambda i: (jnp.maximum(i - _NPB, 0), 0)
    cmap3 = lambda i: (jnp.maximum(i - _NPB, 0), 0, 0)
    full = lambda i: (0, 0)
    return pl.pallas_call(
        _proj_body,
        grid=(_NT // _BP,),
        in_specs=[
            pl.BlockSpec((_BP, 56), pmap),
            pl.BlockSpec((1, 1, _BP), pmap3),
            pl.BlockSpec((_BP, 48), cmap),
            pl.BlockSpec((1, 1, _BP), cmap3),
            pl.BlockSpec((1, 1, _BP), cmap3),
            pl.BlockSpec((50, 8), full),
            pl.BlockSpec((12, 8), full),
            pl.BlockSpec((150, 8), full),
            pl.BlockSpec((_H, _H), full),
            pl.BlockSpec((1, _H), full),
            pl.BlockSpec((_H, _H), full),
            pl.BlockSpec((1, _H), full),
        ],
        out_specs=pl.BlockSpec((_BP, _H), lambda i: (i, 0)),
        out_shape=jax.ShapeDtypeStruct((_NT, _H), jnp.float32),
    )(xp, pidx3, xc, sidx3, iidx3, ptab, stab, itab, wp_t, bp2, wc_t, bc2)


def _layer_body(acc_ref, cnt_ref, h_ref, wl_ref, b_ref, wr_ref, o_ref, *,
                relu):
    cnt = cnt_ref[0] + cnt_ref[1]
    inv = 1.0 / jnp.maximum(cnt[:, :1], 1.0)
    mean = jnp.concatenate([acc_ref[0], acc_ref[1]], axis=1) * inv
    y = (jnp.dot(mean, wl_ref[...], preferred_element_type=jnp.float32)
         + b_ref[...]
         + jnp.dot(h_ref[...], wr_ref[...], preferred_element_type=jnp.float32))
    if relu:
        y = jnp.maximum(y, 0.0)
    o_ref[...] = y


def _layer(acc, cnt, h, wl_t, b2, wr_t, relu):
    grid = _NT // _BP
    return pl.pallas_call(
        functools.partial(_layer_body, relu=relu),
        grid=(grid,),
        in_specs=[
            pl.BlockSpec((2, _BP, _HH), lambda i: (0, i, 0)),
            pl.BlockSpec((2, _BP, _CW), lambda i: (0, i, 0)),
            pl.BlockSpec((_BP, _H), lambda i: (i, 0)),
            pl.BlockSpec((_H, _H), lambda i: (0, 0)),
            pl.BlockSpec((1, _H), lambda i: (0, 0)),
            pl.BlockSpec((_H, _H), lambda i: (0, 0)),
        ],
        out_specs=pl.BlockSpec((_BP, _H), lambda i: (i, 0)),
        out_shape=jax.ShapeDtypeStruct((_NT, _H), jnp.float32),
    )(acc, cnt, h, wl_t, b2, wr_t)


def kernel(x_pol_dyn, pol_state_idx, x_comp_dyn, comp_sector_idx, comp_ind_idx,
           edge_index, state_emb_table, sector_emb_table, ind_emb_table,
           W_pol, b_pol, W_comp, b_comp, W1_l, b1, W1_r, W2_l, b2, W2_r):
    f32 = jnp.float32
    i32 = jnp.int32

    pol_idx3 = pol_state_idx.astype(i32).reshape(_NP // _BP, 1, _BP)
    sec_idx3 = comp_sector_idx.astype(i32).reshape(_NC // _BP, 1, _BP)
    ind_idx3 = comp_ind_idx.astype(i32).reshape(_NC // _BP, 1, _BP)

    h = _proj(x_pol_dyn.astype(f32), pol_idx3, x_comp_dyn.astype(f32),
              sec_idx3, ind_idx3, state_emb_table, sector_emb_table,
              ind_emb_table, W_pol.T.astype(f32), b_pol.reshape(1, _H),
              W_comp.T.astype(f32), b_comp.reshape(1, _H))

    src = edge_index[0].astype(i32)
    dst = edge_index[1].astype(i32)
    pad = _EP - _E
    srcp = jnp.concatenate([src, jnp.zeros((pad,), i32)])
    dstp = jnp.concatenate([dst, jnp.full((pad,), _NT, i32)])
    src2 = srcp.reshape(-1, 128)
    dst2 = dstp.reshape(-1, 128)
    dstc = dstp.reshape(-1, _BLK * 128)

    z32 = jnp.zeros((_RPT, _HH), f32)
    zc = jnp.zeros((_RPT, _CW), f32)
    ones = jnp.ones((_BLK * 128, _CW), f32)

    cnt = _sc_counts(dstc, zc, ones)
    acc1 = _sc_segsum_plain(h.reshape(-1, _HH), src2, dst2, z32)
    h1 = _layer(acc1, cnt, h, W1_l.T.astype(f32), b1.reshape(1, _H),
                W1_r.T.astype(f32), relu=True)

    acc2 = _sc_segsum_plain(h1.reshape(-1, _HH), src2, dst2, z32)
    h2 = _layer(acc2, cnt, h1, W2_l.T.astype(f32), b2.reshape(1, _H),
                W2_r.T.astype(f32), relu=False)

    return (h2[:_NP], h2[_NP:])


# speedup vs baseline: 1.3179x; 1.0093x over previous
import functools

import jax
import jax.numpy as jnp
from jax import lax
from jax.experimental import pallas as pl
from jax.experimental.pallas import tpu as pltpu
from jax.experimental.pallas import tpu_sc as plsc

_NP = 10000
_NC = 40000
_NT = _NP + _NC
_E = 800000
_H = 64
_HH = 32
_CW = 8

_R = 50016
_TG = 392
_BLK = 7
_NBLK = _TG // _BLK
_EP = 16 * _TG * 128
_RPT = _R // 16


def _sc_mesh():
    return plsc.VectorSubcoreMesh(
        core_axis_name="c", subcore_axis_name="s", num_cores=2, num_subcores=16
    )


def _make_sc_segsum():
    out_type = jax.ShapeDtypeStruct((2, _R, _HH), jnp.float32)
    scratch = [
        pltpu.VMEM((_BLK, 128), jnp.int32),
        pltpu.VMEM((_BLK, 128), jnp.int32),
        pltpu.VMEM((_BLK, 128, _HH), jnp.float32),
        pltpu.VMEM_SHARED((_R, _HH), jnp.float32),
        pltpu.SemaphoreType.DMA,
        pltpu.SemaphoreType.DMA,
    ]

    def body(h_ref, srcb_ref, dstb_ref, z32_ref, acc_out,
             src_b, dst_b, rows_b, acc_sp, semg, sems):
        c = lax.axis_index("c")
        s = lax.axis_index("s")
        rb = s * _RPT

        pltpu.sync_copy(z32_ref, acc_sp.at[pl.ds(rb, _RPT)])
        plsc.subcore_barrier()

        gbase = s * _TG

        def blk_body(b, carry):
            g0 = gbase + b * _BLK
            pltpu.sync_copy(srcb_ref.at[pl.ds(g0, _BLK)], src_b)
            pltpu.sync_copy(dstb_ref.at[pl.ds(g0, _BLK)], dst_b)
            for gi in range(_BLK):
                for j in range(8):
                    v = src_b[gi, pl.ds(j * 16, 16)]
                    src_b[gi, pl.ds(j * 16, 16)] = v * 2 + c
            gs = [
                pltpu.async_copy(h_ref.at[src_b.at[gi]], rows_b.at[gi], semg)
                for gi in range(_BLK)
            ]
            for cp in gs:
                cp.wait()
            ss = [
                pltpu.async_copy(
                    rows_b.at[gi], acc_sp.at[dst_b.at[gi]], sems, add=True)
                for gi in range(_BLK)
            ]
            for cp in ss:
                cp.wait()
            return carry

        lax.fori_loop(0, _NBLK, blk_body, 0)
        plsc.subcore_barrier()

        pltpu.sync_copy(acc_sp.at[pl.ds(rb, _RPT)], acc_out.at[c, pl.ds(rb, _RPT)])

    return pl.kernel(
        body, out_type=out_type, mesh=_sc_mesh(), scratch_types=scratch,
        compiler_params=pltpu.CompilerParams(use_tc_tiling_on_sc=False),
    )


def _make_sc_counts():
    blk_e = _BLK * 128
    out_type = jax.ShapeDtypeStruct((2, _R, _CW), jnp.float32)
    scratch = [
        pltpu.VMEM((blk_e,), jnp.int32),
        pltpu.VMEM((blk_e, _CW), jnp.float32),
        pltpu.VMEM_SHARED((_R, _CW), jnp.float32),
    ]
    half_b = _NBLK // 2

    def body(dstb_ref, zc_ref, ones_hbm, cnt_out, dst_b, ones_b, cnt_sp):
        c = lax.axis_index("c")
        s = lax.axis_index("s")
        rb = s * _RPT

        pltpu.sync_copy(zc_ref, cnt_sp.at[pl.ds(rb, _RPT)])
        pltpu.sync_copy(ones_hbm, ones_b)
        plsc.subcore_barrier()

        bbase = s * _NBLK + c * half_b

        def blk_body(b, carry):
            pltpu.sync_copy(dstb_ref.at[bbase + b], dst_b)
            pltpu.sync_copy(ones_b, cnt_sp.at[dst_b], add=True)
            return carry

        lax.fori_loop(0, half_b, blk_body, 0)
        plsc.subcore_barrier()

        pltpu.sync_copy(cnt_sp.at[pl.ds(rb, _RPT)], cnt_out.at[c, pl.ds(rb, _RPT)])

    return pl.kernel(
        body, out_type=out_type, mesh=_sc_mesh(), scratch_types=scratch,
        compiler_params=pltpu.CompilerParams(use_tc_tiling_on_sc=False),
    )


_sc_segsum_plain = _make_sc_segsum()
_sc_counts = _make_sc_counts()


_BP = 1000


_NPB = _NP // _BP


def _proj_body(xp_ref, pidx_ref, xc_ref, sidx_ref, iidx_ref,
               ptab_ref, stab_ref, itab_ref,
               wp_ref, bp_ref, wc_ref, bc_ref, o_ref):
    pid = pl.program_id(0)

    @pl.when(pid < _NPB)
    def _():
        idx = pidx_ref[0, 0, :]
        oh = (idx[:, None] == lax.broadcasted_iota(jnp.int32, (_BP, 50), 1)
              ).astype(jnp.float32)
        emb = jnp.dot(oh, ptab_ref[...], preferred_element_type=jnp.float32)
        feat = jnp.concatenate([xp_ref[...], emb], axis=1)
        h = (jnp.dot(feat, wp_ref[...], preferred_element_type=jnp.float32)
             + bp_ref[...])
        o_ref[...] = jnp.maximum(h, 0.0)

    @pl.when(pid >= _NPB)
    def _():
        sidx = sidx_ref[0, 0, :]
        iidx = iidx_ref[0, 0, :]
        soh = (sidx[:, None] == lax.broadcasted_iota(jnp.int32, (_BP, 12), 1)
               ).astype(jnp.float32)
        ioh = (iidx[:, None] == lax.broadcasted_iota(jnp.int32, (_BP, 150), 1)
               ).astype(jnp.float32)
        semb = jnp.dot(soh, stab_ref[...], preferred_element_type=jnp.float32)
        iemb = jnp.dot(ioh, itab_ref[...], preferred_element_type=jnp.float32)
        feat = jnp.concatenate([xc_ref[...], semb, iemb], axis=1)
        h = (jnp.dot(feat, wc_ref[...], preferred_element_type=jnp.float32)
             + bc_ref[...])
        o_ref[...] = jnp.maximum(h, 0.0)


def _proj(xp, pidx3, xc, sidx3, iidx3, ptab, stab, itab, wp_t, bp2, wc_t, bc2):
    pmap = lambda i: (jnp.minimum(i, _NPB - 1), 0)
    pmap3 = lambda i: (jnp.minimum(i, _NPB - 1), 0, 0)
    cmap = lambda i: (jnp.maximum(i - _NPB, 0), 0)
    cmap3 = lambda i: (jnp.maximum(i - _NPB, 0), 0, 0)
    full = lambda i: (0, 0)
    return pl.pallas_call(
        _proj_body,
        grid=(_NT // _BP,),
        in_specs=[
            pl.BlockSpec((_BP, 56), pmap),
            pl.BlockSpec((1, 1, _BP), pmap3),
            pl.BlockSpec((_BP, 48), cmap),
            pl.BlockSpec((1, 1, _BP), cmap3),
            pl.BlockSpec((1, 1, _BP), cmap3),
            pl.BlockSpec((50, 8), full),
            pl.BlockSpec((12, 8), full),
            pl.BlockSpec((150, 8), full),
            pl.BlockSpec((_H, _H), full),
            pl.BlockSpec((1, _H), full),
            pl.BlockSpec((_H, _H), full),
            pl.BlockSpec((1, _H), full),
        ],
        out_specs=pl.BlockSpec((_BP, _H), lambda i: (i, 0)),
        out_shape=jax.ShapeDtypeStruct((_NT, _H), jnp.float32),
    )(xp, pidx3, xc, sidx3, iidx3, ptab, stab, itab, wp_t, bp2, wc_t, bc2)


def _layer_body(acc_ref, cnt_ref, h_ref, wl_ref, b_ref, wr_ref, o_ref, *,
                relu):
    cnt = cnt_ref[0] + cnt_ref[1]
    inv = 1.0 / jnp.maximum(cnt[:, :1], 1.0)
    mean = jnp.concatenate([acc_ref[0], acc_ref[1]], axis=1) * inv
    y = (jnp.dot(mean, wl_ref[...], preferred_element_type=jnp.float32)
         + b_ref[...]
         + jnp.dot(h_ref[...], wr_ref[...], preferred_element_type=jnp.float32))
    if relu:
        y = jnp.maximum(y, 0.0)
    o_ref[...] = y


def _layer(acc, cnt, h, wl_t, b2, wr_t, relu):
    grid = _NT // _BP
    return pl.pallas_call(
        functools.partial(_layer_body, relu=relu),
        grid=(grid,),
        in_specs=[
            pl.BlockSpec((2, _BP, _HH), lambda i: (0, i, 0)),
            pl.BlockSpec((2, _BP, _CW), lambda i: (0, i, 0)),
            pl.BlockSpec((_BP, _H), lambda i: (i, 0)),
            pl.BlockSpec((_H, _H), lambda i: (0, 0)),
            pl.BlockSpec((1, _H), lambda i: (0, 0)),
            pl.BlockSpec((_H, _H), lambda i: (0, 0)),
        ],
        out_specs=pl.BlockSpec((_BP, _H), lambda i: (i, 0)),
        out_shape=jax.ShapeDtypeStruct((_NT, _H), jnp.float32),
    )(acc, cnt, h, wl_t, b2, wr_t)


def kernel(x_pol_dyn, pol_state_idx, x_comp_dyn, comp_sector_idx, comp_ind_idx,
           edge_index, state_emb_table, sector_emb_table, ind_emb_table,
           W_pol, b_pol, W_comp, b_comp, W1_l, b1, W1_r, W2_l, b2, W2_r):
    f32 = jnp.float32
    i32 = jnp.int32

    pol_idx3 = pol_state_idx.astype(i32).reshape(_NP // _BP, 1, _BP)
    sec_idx3 = comp_sector_idx.astype(i32).reshape(_NC // _BP, 1, _BP)
    ind_idx3 = comp_ind_idx.astype(i32).reshape(_NC // _BP, 1, _BP)

    h = _proj(x_pol_dyn.astype(f32), pol_idx3, x_comp_dyn.astype(f32),
              sec_idx3, ind_idx3, state_emb_table, sector_emb_table,
              ind_emb_table, W_pol.T.astype(f32), b_pol.reshape(1, _H),
              W_comp.T.astype(f32), b_comp.reshape(1, _H))

    src = edge_index[0].astype(i32)
    dst = edge_index[1].astype(i32)
    pad = _EP - _E
    srcp = jnp.concatenate([src, jnp.zeros((pad,), i32)])
    dstp = jnp.concatenate([dst, jnp.full((pad,), _NT, i32)])
    src2 = srcp.reshape(-1, 128)
    dst2 = dstp.reshape(-1, 128)
    dstc = dstp.reshape(-1, _BLK * 128)

    z32 = jnp.zeros((_RPT, _HH), f32)
    zc = jnp.zeros((_RPT, _CW), f32)
    ones = jnp.ones((_BLK * 128, _CW), f32)

    cnt = _sc_counts(dstc, zc, ones)
    acc1 = _sc_segsum_plain(h.reshape(-1, _HH), src2, dst2, z32)
    h1 = _layer(acc1, cnt, h, W1_l.T.astype(f32), b1.reshape(1, _H),
                W1_r.T.astype(f32), relu=True)

    acc2 = _sc_segsum_plain(h1.reshape(-1, _HH), src2, dst2, z32)
    h2 = _layer(acc2, cnt, h1, W2_l.T.astype(f32), b2.reshape(1, _H),
                W2_r.T.astype(f32), relu=False)

    return (h2[:_NP], h2[_NP:])
